# baseline (speedup 1.0000x reference)
import jax
import jax.numpy as jnp
from jax.experimental import pallas as pl
from jax.experimental.pallas import tpu as pltpu
from jax.experimental.pallas import tpu_sc as plsc

_N, _D, _K = 4096, 256, 8192
_TM = 512
_W = 128
_FIX = 128
_MARGIN = 1.0 + 2.0 ** -18
_NB = 4


def _phase_a(x2, csq_ref, cb_ref, u_s, bu_s):
    x_sq = jnp.sum(x2 * x2, axis=1, keepdims=True) * 0.25
    kb = _K // _NB
    ms = []
    for b in range(_NB):
        sl = pl.ds(b * kb, kb)
        cross2 = jax.lax.dot_general(
            x2, cb_ref[sl, :], (((1,), (1,)), ((), ())),
            preferred_element_type=jnp.float32)
        u = csq_ref[:, sl] - cross2
        u_s[:, sl] = u
        ms.append(jnp.min(u, axis=1, keepdims=True))
    mu = jnp.minimum(jnp.minimum(ms[0], ms[1]), jnp.minimum(ms[2], ms[3]))
    bu_s[...] = jnp.maximum(x_sq + mu, 1e-12) * _MARGIN - x_sq


def _phase_b(iota_ref, rhs_ref, u_s, bu_s):
    kb = _K // _NB
    bu = bu_s[...]
    acc = jnp.zeros((_TM, 1), jnp.int32)
    for b in range(_NB - 1):
        sl = pl.ds(b * kb, kb)
        iota = jnp.broadcast_to(iota_ref[:, sl], (_TM, kb))
        acc = acc + jnp.sum(jnp.where(u_s[:, sl] <= bu, iota, 0),
                            axis=1, keepdims=True)
    sl = pl.ds((_NB - 1) * kb, kb)
    q = jnp.where(u_s[:, sl] <= bu, 1.0, 0.0)
    ext = jax.lax.dot_general(
        q, rhs_ref[...], (((1,), (0,)), ((), ())),
        preferred_element_type=jnp.float32)
    return acc[:, 0], ext


def _cand_kernel(csq_ref, x2_ref, cb_ref, iota_ref, rhs_ref,
                 acc_ref, ext_ref, ua, ub, bua, bub):
    acc_a, ext_a = _phase_b(iota_ref, rhs_ref, ua, bua)
    acc_ref[0, 0, 0:_TM] = acc_a
    ext_ref[0:_TM, :] = ext_a
    _phase_a(x2_ref[0:_TM, :], csq_ref, cb_ref, ua, bua)
    acc_b, ext_b = _phase_b(iota_ref, rhs_ref, ub, bub)
    acc_ref[0, 0, _TM:2 * _TM] = acc_b
    ext_ref[_TM:2 * _TM, :] = ext_b
    _phase_a(x2_ref[_TM:2 * _TM, :], csq_ref, cb_ref, ub, bub)


def _fix_kernel(csq_ref, xsq_ref, x_ref, cb_ref, idx_ref):
    cross = jax.lax.dot_general(
        x_ref[...], cb_ref[...], (((1,), (1,)), ((), ())),
        preferred_element_type=jnp.float32)
    t = xsq_ref[...] + csq_ref[...] - 2.0 * cross
    dist = jnp.sqrt(jnp.maximum(t, 1e-12))
    idx_ref[0, 0, :] = jnp.argmin(dist, axis=1).astype(jnp.int32)


def _sc_gather(table, indices):
    mesh = plsc.VectorSubcoreMesh(
        core_axis_name="core", subcore_axis_name="subcore")
    i2 = indices.reshape(1, _N)

    @pl.kernel(out_type=jax.ShapeDtypeStruct((_N, _D), table.dtype),
               mesh=mesh)
    def gk(tab_hbm, i_hbm, o_hbm):
        def body(i_vmem, o_vmem):
            pltpu.sync_copy(tab_hbm.at[i_vmem.at[0]], o_vmem)

        pltpu.emit_pipeline(
            body,
            grid=(_N // _W,),
            in_specs=[pl.BlockSpec((1, _W), lambda i: (0, i))],
            out_specs=[pl.BlockSpec((_W, _D), lambda i: (i, 0))],
            core_axis_name=("core", "subcore"),
            dimension_semantics=(pltpu.PARALLEL,),
        )(i_hbm, o_hbm)

    return gk(table, i2)


def kernel(x, codebook, embedding_table):
    csq = jnp.sum(codebook * codebook, axis=-1)[None, :]
    iota1 = (16384 + jnp.arange(_K, dtype=jnp.int32)).reshape(1, _K)
    jlast = jnp.arange(_K - _K // _NB, _K, dtype=jnp.int32)
    rhs = jnp.zeros((_K // _NB, 128), jnp.float32)
    rhs = rhs.at[:, 0].set(1.0)
    rhs = rhs.at[:, 1].set((jlast // 64).astype(jnp.float32))
    rhs = rhs.at[:, 2].set((jlast % 64).astype(jnp.float32))
    x2 = x + x
    npair = _N // (2 * _TM)

    acc, ext = pl.pallas_call(
        _cand_kernel,
        grid=(npair + 1,),
        in_specs=[
            pl.BlockSpec((1, _K), lambda i: (0, 0)),
            pl.BlockSpec((2 * _TM, _D),
                         lambda i: (jnp.minimum(i, npair - 1), 0)),
            pl.BlockSpec((_K, _D), lambda i: (0, 0)),
            pl.BlockSpec((1, _K), lambda i: (0, 0)),
            pl.BlockSpec((_K // _NB, 128), lambda i: (0, 0)),
        ],
        out_specs=[
            pl.BlockSpec((1, 1, 2 * _TM),
                         lambda i: (jnp.maximum(i - 1, 0), 0, 0)),
            pl.BlockSpec((2 * _TM, 128),
                         lambda i: (jnp.maximum(i - 1, 0), 0)),
        ],
        out_shape=[
            jax.ShapeDtypeStruct((npair, 1, 2 * _TM), jnp.int32),
            jax.ShapeDtypeStruct((_N, 128), jnp.float32),
        ],
        scratch_shapes=[
            pltpu.VMEM((_TM, _K), jnp.float32),
            pltpu.VMEM((_TM, _K), jnp.float32),
            pltpu.VMEM((_TM, 1), jnp.float32),
            pltpu.VMEM((_TM, 1), jnp.float32),
        ],
        compiler_params=pltpu.CompilerParams(
            dimension_semantics=("arbitrary",)),
    )(csq, x2, codebook, iota1, rhs)

    acc = acc.reshape(_N)
    cnt_a = acc >> 14
    cnt_b = ext[:, 0].astype(jnp.int32)
    j1 = jnp.where(cnt_a == 1, acc & 16383,
                   (64.0 * ext[:, 1] + ext[:, 2]).astype(jnp.int32))
    flagged = (cnt_a + cnt_b) != 1
    fix_rows = jnp.where(flagged, size=_FIX, fill_value=0)[0]
    x_fix = x[fix_rows]
    xsq_fix = jnp.sum(x_fix * x_fix, axis=-1, keepdims=True)
    fixed = pl.pallas_call(
        _fix_kernel,
        grid=(1,),
        in_specs=[
            pl.BlockSpec((1, _K), lambda i: (0, 0)),
            pl.BlockSpec((_FIX, 1), lambda i: (0, 0)),
            pl.BlockSpec((_FIX, _D), lambda i: (0, 0)),
            pl.BlockSpec((_K, _D), lambda i: (0, 0)),
        ],
        out_specs=pl.BlockSpec((1, 1, _FIX), lambda i: (0, 0, 0)),
        out_shape=jax.ShapeDtypeStruct((1, 1, _FIX), jnp.int32),
    )(csq, xsq_fix, x_fix, codebook).reshape(_FIX)

    indices = j1.at[fix_rows].set(fixed)
    return _sc_gather(embedding_table, indices)


# speedup vs baseline: 1.1408x; 1.1408x over previous
import jax
import jax.numpy as jnp
from jax.experimental import pallas as pl
from jax.experimental.pallas import tpu as pltpu
from jax.experimental.pallas import tpu_sc as plsc

_N, _D, _K = 4096, 256, 8192
_TM = 256
_W = 128
_FIX = 256
_MARGIN = 1.0 + 2.0 ** -18
_NB = 4


def _cand_kernel(csq_ref, x_ref, cb_ref, iota_ref, idx_ref):
    x = x_ref[...]
    x_sq = jnp.sum(x * x, axis=1, keepdims=True)
    kb = _K // _NB
    ts, ms = [], []
    for b in range(_NB):
        sl = pl.ds(b * kb, kb)
        cross = jax.lax.dot_general(
            x, cb_ref[sl, :], (((1,), (1,)), ((), ())),
            preferred_element_type=jnp.float32)
        t = x_sq + csq_ref[:, sl] - 2.0 * cross
        ts.append(t)
        ms.append(jnp.min(t, axis=1, keepdims=True))
    m = jnp.minimum(jnp.minimum(ms[0], ms[1]), jnp.minimum(ms[2], ms[3]))
    bthr = jnp.maximum(m, 1e-12) * _MARGIN
    acc = jnp.zeros((_TM, 1), jnp.int32)
    for b in range(_NB):
        sl = pl.ds(b * kb, kb)
        iota = jnp.broadcast_to(iota_ref[:, sl], (_TM, kb))
        acc = acc + jnp.sum(jnp.where(ts[b] <= bthr, iota, 0),
                            axis=1, keepdims=True)
    idx = jnp.where(acc < 32768, acc - 16384, -1)
    idx_ref[0, 0, :] = idx[:, 0]


def _fix_kernel(csq_ref, xsq_ref, x_ref, cb_ref, idx_ref):
    cross = jax.lax.dot_general(
        x_ref[...], cb_ref[...], (((1,), (1,)), ((), ())),
        preferred_element_type=jnp.float32)
    t = xsq_ref[...] + csq_ref[...] - 2.0 * cross
    dist = jnp.sqrt(jnp.maximum(t, 1e-12))
    idx_ref[0, 0, :] = jnp.argmin(dist, axis=1).astype(jnp.int32)


def _sc_gather(table, indices):
    mesh = plsc.VectorSubcoreMesh(
        core_axis_name="core", subcore_axis_name="subcore")
    i2 = indices.reshape(1, _N)

    @pl.kernel(out_type=jax.ShapeDtypeStruct((_N, _D), table.dtype),
               mesh=mesh)
    def gk(tab_hbm, i_hbm, o_hbm):
        def body(i_vmem, o_vmem):
            pltpu.sync_copy(tab_hbm.at[i_vmem.at[0]], o_vmem)

        pltpu.emit_pipeline(
            body,
            grid=(_N // _W,),
            in_specs=[pl.BlockSpec((1, _W), lambda i: (0, i))],
            out_specs=[pl.BlockSpec((_W, _D), lambda i: (i, 0))],
            core_axis_name=("core", "subcore"),
            dimension_semantics=(pltpu.PARALLEL,),
        )(i_hbm, o_hbm)

    return gk(table, i2)


def kernel(x, codebook, embedding_table):
    csq = jnp.sum(codebook * codebook, axis=-1)[None, :]
    xsq = jnp.sum(x * x, axis=-1, keepdims=True)
    iota1 = (16384 + jnp.arange(_K, dtype=jnp.int32)).reshape(1, _K)

    j1 = pl.pallas_call(
        _cand_kernel,
        grid=(_N // _TM,),
        in_specs=[
            pl.BlockSpec((1, _K), lambda i: (0, 0)),
            pl.BlockSpec((_TM, _D), lambda i: (i, 0)),
            pl.BlockSpec((_K, _D), lambda i: (0, 0)),
            pl.BlockSpec((1, _K), lambda i: (0, 0)),
        ],
        out_specs=pl.BlockSpec((1, 1, _TM), lambda i: (i, 0, 0)),
        out_shape=jax.ShapeDtypeStruct((_N // _TM, 1, _TM), jnp.int32),
        compiler_params=pltpu.CompilerParams(
            dimension_semantics=("arbitrary",)),
    )(csq, x, codebook, iota1).reshape(_N)

    flagged = j1 < 0
    fix_rows = jnp.where(flagged, size=_FIX, fill_value=0)[0]
    x_fix = x[fix_rows]
    xsq_fix = xsq[fix_rows]
    d2_fix = jnp.maximum(
        xsq_fix + csq - 2.0 * (x_fix @ codebook.T), 0.0)
    fixed = jnp.argmin(jnp.sqrt(jnp.maximum(d2_fix, 1e-12)), axis=-1)

    indices = j1.at[fix_rows].set(fixed.astype(jnp.int32))
    return _sc_gather(embedding_table, indices)
